# baseline (speedup 1.0000x reference)
import functools

import jax
import jax.numpy as jnp
from jax import lax
from jax.experimental import pallas as pl
from jax.experimental.pallas import tpu as pltpu
from jax.experimental.pallas import tpu_sc as plsc

B = 8192
K = 8192
D = 256
BETA = 0.25

BB = 1024
KB = 4096
NB = B // BB
NK = K // KB
RG = 128
NCH = KB // 128
BIG = 2**30


def _argmin_body(x_ref, cb_ref, idx_ref, runmin_ref, runidx_ref):
    k = pl.program_id(1)
    sim = lax.dot_general(x_ref[...], cb_ref[...],
                          dimension_numbers=(((1,), (1,)), ((), ())),
                          preferred_element_type=jnp.float32)
    rowm_parts = []
    ci_parts = []
    for rg in range(BB // RG):
        rs = slice(rg * RG, (rg + 1) * RG)
        m = 1.0 - sim[rs, 0:128]
        ci = jnp.zeros((RG, 128), jnp.float32)
        for c in range(1, NCH):
            dc = 1.0 - sim[rs, c * 128:(c + 1) * 128]
            lt = dc < m
            m = jnp.where(lt, dc, m)
            ci = jnp.where(lt, jnp.float32(c), ci)
        rowm_parts.append(m)
        ci_parts.append(ci)
    mv = jnp.concatenate(rowm_parts, axis=0)
    civ = jnp.concatenate(ci_parts, axis=0).astype(jnp.int32)
    lane = lax.broadcasted_iota(jnp.int32, (BB, 128), 1)
    rowm = jnp.min(mv, axis=1, keepdims=True)
    j = (k * NCH + civ) * 128 + lane
    jbest = jnp.min(jnp.where(mv == rowm, j, BIG), axis=1, keepdims=True)

    @pl.when(k == 0)
    def _first_half():
        runmin_ref[...] = rowm.astype(jnp.bfloat16).astype(jnp.float32)
        runidx_ref[...] = jbest

    @pl.when(k == NK - 1)
    def _second_half():
        better = rowm < runmin_ref[...]
        idx_ref[...] = jnp.where(better, jbest,
                                 runidx_ref[...]).reshape(1, 1, BB)


def _argmin_call(x, cb):
    return pl.pallas_call(
        _argmin_body,
        grid=(NB, NK),
        in_specs=[
            pl.BlockSpec((BB, D), lambda b, k: (b, 0)),
            pl.BlockSpec((KB, D), lambda b, k: (k, 0)),
        ],
        out_specs=pl.BlockSpec((1, 1, BB), lambda b, k: (b, 0, 0)),
        out_shape=jax.ShapeDtypeStruct((NB, 1, BB), jnp.int32),
        scratch_shapes=[
            pltpu.VMEM((BB, 1), jnp.float32),
            pltpu.VMEM((BB, 1), jnp.int32),
        ],
    )(x, cb)


NC = 2
NS = 16
NW = NC * NS
BPW = B // NW
GCH = 128
NGC = BPW // GCH


def _gather_body(table_hbm, idx_hbm, out_hbm, idx_v, rows_v, sem):
    wid = lax.axis_index("s") * NC + lax.axis_index("c")
    pltpu.sync_copy(idx_hbm.at[pl.ds(wid * NGC, NGC)], idx_v)
    copies = [
        pltpu.async_copy(table_hbm.at[idx_v.at[j]],
                         rows_v.at[pl.ds(j * GCH, GCH)], sem)
        for j in range(NGC)
    ]
    for c in copies:
        c.wait()
    pltpu.sync_copy(rows_v, out_hbm.at[pl.ds(wid * BPW, BPW)])


def _gather_call(cb, idx):
    idx2 = idx.reshape(NW * NGC, GCH)
    f = functools.partial(
        pl.kernel,
        mesh=plsc.VectorSubcoreMesh(core_axis_name="c", subcore_axis_name="s"),
        out_type=jax.ShapeDtypeStruct((B, D), jnp.float32),
        scratch_types=[
            pltpu.VMEM((NGC, GCH), jnp.int32),
            pltpu.VMEM((BPW, D), jnp.float32),
            pltpu.SemaphoreType.DMA,
        ],
    )(_gather_body)
    return f(cb, idx2)


RB = 2048
NR = B // RB


def _proj_body(x_ref, cb_ref, xq_ref, scal_ref, loss_ref, acc_ref):
    i = pl.program_id(0)
    xx = x_ref[...]
    cb = cb_ref[...]
    dot = jnp.sum(xx * cb, axis=1, keepdims=True)
    nsq = jnp.sum(cb * cb, axis=1, keepdims=True)
    scalar = dot / (nsq + 1e-8)
    proj = scalar * cb
    xq_ref[...] = xx + (proj - xx)
    scal_ref[...] = scalar
    dp = jnp.sum(proj * xx, axis=1)
    npn = jnp.sqrt(jnp.sum(proj * proj, axis=1))
    nx = jnp.sqrt(jnp.sum(xx * xx, axis=1))
    cos = dp / (jnp.maximum(npn, 1e-8) * jnp.maximum(nx, 1e-8))
    part = jnp.sum(1.0 - cos)

    @pl.when(i == 0)
    def _init():
        acc_ref[0, 0] = 0.0

    acc_ref[0, 0] += part

    @pl.when(i == pl.num_programs(0) - 1)
    def _fin():
        loss_ref[...] = jnp.full((1, 1), BETA * (acc_ref[0, 0] / B),
                                 jnp.float32)


def _proj_call(x, cb_vec):
    return pl.pallas_call(
        _proj_body,
        grid=(NR,),
        in_specs=[
            pl.BlockSpec((RB, D), lambda i: (i, 0)),
            pl.BlockSpec((RB, D), lambda i: (i, 0)),
        ],
        out_specs=[
            pl.BlockSpec((RB, D), lambda i: (i, 0)),
            pl.BlockSpec((RB, 1), lambda i: (i, 0)),
            pl.BlockSpec((1, 1), lambda i: (0, 0)),
        ],
        out_shape=[
            jax.ShapeDtypeStruct((B, D), jnp.float32),
            jax.ShapeDtypeStruct((B, 1), jnp.float32),
            jax.ShapeDtypeStruct((1, 1), jnp.float32),
        ],
        scratch_shapes=[pltpu.SMEM((1, 1), jnp.float32)],
    )(x, cb_vec)


def kernel(x, embedding_weight):
    xn = x / jnp.maximum(jnp.linalg.norm(x, axis=1, keepdims=True), 1e-12)
    cbn = embedding_weight / jnp.maximum(
        jnp.linalg.norm(embedding_weight, axis=1, keepdims=True), 1e-12)
    idx = _argmin_call(xn, cbn).reshape(B)
    cb_vec = _gather_call(embedding_weight, idx)
    x_q, scal, loss = _proj_call(x, cb_vec)
    return (x_q, loss.reshape(()), idx, scal.reshape(B))


# speedup vs baseline: 1.0046x; 1.0046x over previous
import functools

import jax
import jax.numpy as jnp
from jax import lax
from jax.experimental import pallas as pl
from jax.experimental.pallas import tpu as pltpu
from jax.experimental.pallas import tpu_sc as plsc

B = 8192
K = 8192
D = 256
BETA = 0.25

BB = 1024
KB = 4096
NB = B // BB
NK = K // KB
RG = 128
NCH = KB // 128
BIG = 2**30


def _argmin_body(x_ref, cb_ref, idx_ref, runmin_ref, runidx_ref):
    k = pl.program_id(0)
    b = pl.program_id(1)
    sim = lax.dot_general(x_ref[...], cb_ref[...],
                          dimension_numbers=(((1,), (1,)), ((), ())),
                          preferred_element_type=jnp.float32)
    rowm_parts = []
    ci_parts = []
    for rg in range(BB // RG):
        rs = slice(rg * RG, (rg + 1) * RG)
        m = 1.0 - sim[rs, 0:128]
        ci = jnp.zeros((RG, 128), jnp.float32)
        for c in range(1, NCH):
            dc = 1.0 - sim[rs, c * 128:(c + 1) * 128]
            lt = dc < m
            m = jnp.where(lt, dc, m)
            ci = jnp.where(lt, jnp.float32(c), ci)
        rowm_parts.append(m)
        ci_parts.append(ci)
    mv = jnp.concatenate(rowm_parts, axis=0)
    civ = jnp.concatenate(ci_parts, axis=0).astype(jnp.int32)
    lane = lax.broadcasted_iota(jnp.int32, (BB, 128), 1)
    rowm = jnp.min(mv, axis=1, keepdims=True)
    j = (k * NCH + civ) * 128 + lane
    jbest = jnp.min(jnp.where(mv == rowm, j, BIG), axis=1, keepdims=True)

    @pl.when(k == 0)
    def _first_half():
        runmin_ref[b] = rowm.astype(jnp.bfloat16).astype(jnp.float32)
        runidx_ref[b] = jbest

    @pl.when(k == NK - 1)
    def _second_half():
        better = rowm < runmin_ref[b]
        idx_ref[...] = jnp.where(better, jbest,
                                 runidx_ref[b]).reshape(1, 1, BB)


def _argmin_call(x, cb):
    return pl.pallas_call(
        _argmin_body,
        grid=(NK, NB),
        in_specs=[
            pl.BlockSpec((BB, D), lambda k, b: (b, 0)),
            pl.BlockSpec((KB, D), lambda k, b: (k, 0)),
        ],
        out_specs=pl.BlockSpec((1, 1, BB), lambda k, b: (b, 0, 0)),
        out_shape=jax.ShapeDtypeStruct((NB, 1, BB), jnp.int32),
        scratch_shapes=[
            pltpu.VMEM((NB, BB, 1), jnp.float32),
            pltpu.VMEM((NB, BB, 1), jnp.int32),
        ],
    )(x, cb)


NC = 2
NS = 16
NW = NC * NS
BPW = B // NW
GCH = 128
NGC = BPW // GCH


def _gather_body(table_hbm, idx_hbm, out_hbm, idx_v, rows_v, sem):
    wid = lax.axis_index("s") * NC + lax.axis_index("c")
    pltpu.sync_copy(idx_hbm.at[pl.ds(wid * NGC, NGC)], idx_v)
    copies = [
        pltpu.async_copy(table_hbm.at[idx_v.at[j]],
                         rows_v.at[pl.ds(j * GCH, GCH)], sem)
        for j in range(NGC)
    ]
    for c in copies:
        c.wait()
    pltpu.sync_copy(rows_v, out_hbm.at[pl.ds(wid * BPW, BPW)])


def _gather_call(cb, idx):
    idx2 = idx.reshape(NW * NGC, GCH)
    f = functools.partial(
        pl.kernel,
        mesh=plsc.VectorSubcoreMesh(core_axis_name="c", subcore_axis_name="s"),
        out_type=jax.ShapeDtypeStruct((B, D), jnp.float32),
        scratch_types=[
            pltpu.VMEM((NGC, GCH), jnp.int32),
            pltpu.VMEM((BPW, D), jnp.float32),
            pltpu.SemaphoreType.DMA,
        ],
    )(_gather_body)
    return f(cb, idx2)


RB = 2048
NR = B // RB


def _proj_body(x_ref, cb_ref, xq_ref, scal_ref, loss_ref, acc_ref):
    i = pl.program_id(0)
    xx = x_ref[...]
    cb = cb_ref[...]
    dot = jnp.sum(xx * cb, axis=1, keepdims=True)
    nsq = jnp.sum(cb * cb, axis=1, keepdims=True)
    scalar = dot / (nsq + 1e-8)
    proj = scalar * cb
    xq_ref[...] = xx + (proj - xx)
    scal_ref[...] = scalar
    dp = jnp.sum(proj * xx, axis=1)
    npn = jnp.sqrt(jnp.sum(proj * proj, axis=1))
    nx = jnp.sqrt(jnp.sum(xx * xx, axis=1))
    cos = dp / (jnp.maximum(npn, 1e-8) * jnp.maximum(nx, 1e-8))
    part = jnp.sum(1.0 - cos)

    @pl.when(i == 0)
    def _init():
        acc_ref[0, 0] = 0.0

    acc_ref[0, 0] += part

    @pl.when(i == pl.num_programs(0) - 1)
    def _fin():
        loss_ref[...] = jnp.full((1, 1), BETA * (acc_ref[0, 0] / B),
                                 jnp.float32)


def _proj_call(x, cb_vec):
    return pl.pallas_call(
        _proj_body,
        grid=(NR,),
        in_specs=[
            pl.BlockSpec((RB, D), lambda i: (i, 0)),
            pl.BlockSpec((RB, D), lambda i: (i, 0)),
        ],
        out_specs=[
            pl.BlockSpec((RB, D), lambda i: (i, 0)),
            pl.BlockSpec((RB, 1), lambda i: (i, 0)),
            pl.BlockSpec((1, 1), lambda i: (0, 0)),
        ],
        out_shape=[
            jax.ShapeDtypeStruct((B, D), jnp.float32),
            jax.ShapeDtypeStruct((B, 1), jnp.float32),
            jax.ShapeDtypeStruct((1, 1), jnp.float32),
        ],
        scratch_shapes=[pltpu.SMEM((1, 1), jnp.float32)],
    )(x, cb_vec)


def kernel(x, embedding_weight):
    xn = x / jnp.maximum(jnp.linalg.norm(x, axis=1, keepdims=True), 1e-12)
    cbn = embedding_weight / jnp.maximum(
        jnp.linalg.norm(embedding_weight, axis=1, keepdims=True), 1e-12)
    idx = _argmin_call(xn, cbn).reshape(B)
    cb_vec = _gather_call(embedding_weight, idx)
    x_q, scal, loss = _proj_call(x, cb_vec)
    return (x_q, loss.reshape(()), idx, scal.reshape(B))
